# baseline (speedup 1.0000x reference)
import functools

import jax
import jax.numpy as jnp
from jax import lax
from jax.experimental import pallas as pl
from jax.experimental.pallas import tpu as pltpu
from jax.experimental.pallas import tpu_sc as plsc

FIELDS = 26
DIM = 32
BATCH = 16384
V = FIELDS * 100000
NC, NS, L = 2, 16, 16
NW = NC * NS
SUB = 128
GROUP = 4
UNITS = FIELDS * (BATCH // (SUB * GROUP))
UNITS_W = UNITS // NW
BBLKS = BATCH // (SUB * GROUP)

_mesh = plsc.VectorSubcoreMesh(core_axis_name="c", subcore_axis_name="s")


TBLK = 2048
TGRID = (V + TBLK - 1) // TBLK


def _transpose_block(tin_ref, tout_ref):
    x3 = tin_ref[...].reshape(DIM, TBLK // 4, 4)
    y3 = jnp.transpose(x3, (1, 2, 0))
    tout_ref[...] = y3.reshape(TBLK // 4, 4 * DIM)


_table_transpose = pl.pallas_call(
    _transpose_block,
    grid=(TGRID,),
    in_specs=[pl.BlockSpec((DIM, TBLK), lambda j: (0, j))],
    out_specs=pl.BlockSpec((TBLK // 4, 4 * DIM), lambda j: (j, 0)),
    out_shape=jax.ShapeDtypeStruct((V * DIM // 128, 128), jnp.float32),
)


@functools.partial(
    pl.kernel,
    out_type=jax.ShapeDtypeStruct((FIELDS, BATCH, DIM), jnp.float32),
    mesh=_mesh,
    compiler_params=pltpu.CompilerParams(use_tc_tiling_on_sc=False),
    scratch_types=[
        pltpu.VMEM((3 * L,), jnp.int32),
        pltpu.VMEM((GROUP, SUB), jnp.int32),
        pltpu.VMEM((GROUP * SUB, DIM), jnp.float32),
        pltpu.SemaphoreType.DMA,
    ],
)
def _buckle_gather(idx_hbm, off_hbm, table_hbm, out_hbm,
                   off_v, idx_v, rows_v, sem):
    wid = lax.axis_index("s") * NC + lax.axis_index("c")
    pltpu.sync_copy(off_hbm, off_v)

    def unit_body(c, carry):
        u = wid * UNITS_W + c
        f = u // BBLKS
        jb = (u % BBLKS) * GROUP
        foff = off_v[pl.ds(f, L)][0]
        pltpu.sync_copy(idx_hbm.at[f, pl.ds(jb, GROUP)], idx_v)
        for j in range(GROUP):
            for s in range(SUB // L):
                sl = pl.ds(s * L, L)
                idx_v[j, sl] = idx_v[j, sl] + foff
        copies = [
            pltpu.async_copy(table_hbm.at[idx_v.at[j]],
                             rows_v.at[pl.ds(j * SUB, SUB)], sem)
            for j in range(GROUP)
        ]
        for cp in copies:
            cp.wait()
        pltpu.sync_copy(rows_v, out_hbm.at[f, pl.ds(jb * SUB, GROUP * SUB)])
        return carry

    lax.fori_loop(0, UNITS_W, unit_body, 0)


def kernel(categorical_inputs, embedding_weight, offsets):
    table_rm = _table_transpose(embedding_weight.T)
    idx_fm = (categorical_inputs.astype(jnp.int32).T
              .reshape(FIELDS, BATCH // SUB, SUB))
    off = jnp.pad(offsets[:FIELDS].astype(jnp.int32), (0, 3 * L - FIELDS))
    out2 = _buckle_gather(idx_fm, off, table_rm.reshape(V, DIM))
    return jnp.transpose(out2, (1, 0, 2))


# speedup vs baseline: 8.2026x; 8.2026x over previous
import functools

import jax
import jax.numpy as jnp
from jax import lax
from jax.experimental import pallas as pl
from jax.experimental.pallas import tpu as pltpu
from jax.experimental.pallas import tpu_sc as plsc

FIELDS = 26
DIM = 32
BATCH = 16384
V = FIELDS * 100000
NC, NS, L = 2, 16, 16
NW = NC * NS
SUB = 128
GROUP = 4
UNITS = FIELDS * (BATCH // (SUB * GROUP))
UNITS_W = UNITS // NW
BBLKS = BATCH // (SUB * GROUP)

_mesh = plsc.VectorSubcoreMesh(core_axis_name="c", subcore_axis_name="s")


TBLK = 16384
TGRID = (V + TBLK - 1) // TBLK


def _transpose_block(tin_ref, tout_ref):
    xt = tin_ref[...].T
    tout_ref[...] = jnp.concatenate(
        [xt, jnp.zeros((TBLK, 128 - DIM), jnp.float32)], axis=1)


_table_transpose = pl.pallas_call(
    _transpose_block,
    grid=(TGRID,),
    in_specs=[pl.BlockSpec((DIM, TBLK), lambda j: (0, j))],
    out_specs=pl.BlockSpec((TBLK, 128), lambda j: (j, 0)),
    out_shape=jax.ShapeDtypeStruct((V, 128), jnp.float32),
)


@functools.partial(
    pl.kernel,
    out_type=jax.ShapeDtypeStruct((FIELDS, BATCH, DIM), jnp.float32),
    mesh=_mesh,
    compiler_params=pltpu.CompilerParams(use_tc_tiling_on_sc=False),
    scratch_types=[
        pltpu.VMEM((3 * L,), jnp.int32),
        pltpu.VMEM((GROUP, SUB), jnp.int32),
        pltpu.VMEM((GROUP * SUB, DIM), jnp.float32),
        pltpu.SemaphoreType.DMA,
    ],
)
def _buckle_gather(idx_hbm, off_hbm, table_hbm, out_hbm,
                   off_v, idx_v, rows_v, sem):
    wid = lax.axis_index("s") * NC + lax.axis_index("c")
    pltpu.sync_copy(off_hbm, off_v)

    def unit_body(c, carry):
        u = wid * UNITS_W + c
        f = u // BBLKS
        jb = (u % BBLKS) * GROUP
        foff = off_v[pl.ds(f, L)][0]
        pltpu.sync_copy(idx_hbm.at[f, pl.ds(jb, GROUP)], idx_v)
        for j in range(GROUP):
            for s in range(SUB // L):
                sl = pl.ds(s * L, L)
                idx_v[j, sl] = (idx_v[j, sl] + foff) * 4
        copies = [
            pltpu.async_copy(table_hbm.at[idx_v.at[j]],
                             rows_v.at[pl.ds(j * SUB, SUB)], sem)
            for j in range(GROUP)
        ]
        for cp in copies:
            cp.wait()
        pltpu.sync_copy(rows_v, out_hbm.at[f, pl.ds(jb * SUB, GROUP * SUB)])
        return carry

    lax.fori_loop(0, UNITS_W, unit_body, 0)


def kernel(categorical_inputs, embedding_weight, offsets):
    table_rm = _table_transpose(embedding_weight.T)
    idx_fm = (categorical_inputs.astype(jnp.int32).T
              .reshape(FIELDS, BATCH // SUB, SUB))
    off = jnp.pad(offsets[:FIELDS].astype(jnp.int32), (0, 3 * L - FIELDS))
    out2 = _buckle_gather(idx_fm, off, table_rm.reshape(4 * V, DIM))
    return jnp.transpose(out2, (1, 0, 2))


# speedup vs baseline: 8.3678x; 1.0201x over previous
import functools

import jax
import jax.numpy as jnp
from jax import lax
from jax.experimental import pallas as pl
from jax.experimental.pallas import tpu as pltpu
from jax.experimental.pallas import tpu_sc as plsc

FIELDS = 26
DIM = 32
BATCH = 16384
V = FIELDS * 100000
NC, NS, L = 2, 16, 16
NW = NC * NS
SUB = 128
GROUP = 4
UNITS = FIELDS * (BATCH // (SUB * GROUP))
UNITS_W = UNITS // NW
BBLKS = BATCH // (SUB * GROUP)

_mesh = plsc.VectorSubcoreMesh(core_axis_name="c", subcore_axis_name="s")


TBLK = 32768
TGRID = (V + TBLK - 1) // TBLK


def _transpose_block(tin_ref, tout_ref):
    xt = tin_ref[...].T
    tout_ref[...] = jnp.concatenate(
        [xt, jnp.zeros((TBLK, 128 - DIM), jnp.float32)], axis=1)


_table_transpose = pl.pallas_call(
    _transpose_block,
    grid=(TGRID,),
    in_specs=[pl.BlockSpec((DIM, TBLK), lambda j: (0, j))],
    out_specs=pl.BlockSpec((TBLK, 128), lambda j: (j, 0)),
    out_shape=jax.ShapeDtypeStruct((V, 128), jnp.float32),
)


@functools.partial(
    pl.kernel,
    out_type=jax.ShapeDtypeStruct((FIELDS, BATCH, DIM), jnp.float32),
    mesh=_mesh,
    compiler_params=pltpu.CompilerParams(use_tc_tiling_on_sc=False),
    scratch_types=[
        pltpu.VMEM((3 * L,), jnp.int32),
        pltpu.VMEM((GROUP, SUB), jnp.int32),
        pltpu.VMEM((GROUP * SUB, DIM), jnp.float32),
        pltpu.SemaphoreType.DMA,
    ],
)
def _buckle_gather(idx_hbm, off_hbm, table_hbm, out_hbm,
                   off_v, idx_v, rows_v, sem):
    wid = lax.axis_index("s") * NC + lax.axis_index("c")
    pltpu.sync_copy(off_hbm, off_v)

    def unit_body(c, carry):
        u = wid * UNITS_W + c
        f = u // BBLKS
        jb = (u % BBLKS) * GROUP
        foff = off_v[pl.ds(f, L)][0]
        pltpu.sync_copy(idx_hbm.at[f, pl.ds(jb, GROUP)], idx_v)
        for j in range(GROUP):
            for s in range(SUB // L):
                sl = pl.ds(s * L, L)
                idx_v[j, sl] = (idx_v[j, sl] + foff) * 4
        copies = [
            pltpu.async_copy(table_hbm.at[idx_v.at[j]],
                             rows_v.at[pl.ds(j * SUB, SUB)], sem)
            for j in range(GROUP)
        ]
        for cp in copies:
            cp.wait()
        pltpu.sync_copy(rows_v, out_hbm.at[f, pl.ds(jb * SUB, GROUP * SUB)])
        return carry

    lax.fori_loop(0, UNITS_W, unit_body, 0)


def kernel(categorical_inputs, embedding_weight, offsets):
    table_rm = _table_transpose(embedding_weight.T)
    idx_fm = (categorical_inputs.astype(jnp.int32).T
              .reshape(FIELDS, BATCH // SUB, SUB))
    off = jnp.pad(offsets[:FIELDS].astype(jnp.int32), (0, 3 * L - FIELDS))
    out2 = _buckle_gather(idx_fm, off, table_rm.reshape(4 * V, DIM))
    return jnp.transpose(out2, (1, 0, 2))


# speedup vs baseline: 8.5555x; 1.0224x over previous
import functools

import jax
import jax.numpy as jnp
from jax import lax
from jax.experimental import pallas as pl
from jax.experimental.pallas import tpu as pltpu
from jax.experimental.pallas import tpu_sc as plsc

FIELDS = 26
DIM = 32
BATCH = 16384
V = FIELDS * 100000
NC, NS, L = 2, 16, 16
NW = NC * NS
SUB = 128
GROUP = 8
UNITS = FIELDS * (BATCH // (SUB * GROUP))
UNITS_W = UNITS // NW
BBLKS = BATCH // (SUB * GROUP)

_mesh = plsc.VectorSubcoreMesh(core_axis_name="c", subcore_axis_name="s")


TBLK = 32768
TGRID = (V + TBLK - 1) // TBLK


def _transpose_block(tin_ref, tout_ref):
    xt = tin_ref[...].T
    tout_ref[...] = jnp.concatenate(
        [xt, jnp.zeros((TBLK, 128 - DIM), jnp.float32)], axis=1)


_table_transpose = pl.pallas_call(
    _transpose_block,
    grid=(TGRID,),
    in_specs=[pl.BlockSpec((DIM, TBLK), lambda j: (0, j))],
    out_specs=pl.BlockSpec((TBLK, 128), lambda j: (j, 0)),
    out_shape=jax.ShapeDtypeStruct((V, 128), jnp.float32),
)


@functools.partial(
    pl.kernel,
    out_type=jax.ShapeDtypeStruct((FIELDS, BATCH, DIM), jnp.float32),
    mesh=_mesh,
    compiler_params=pltpu.CompilerParams(use_tc_tiling_on_sc=False),
    scratch_types=[
        pltpu.VMEM((3 * L,), jnp.int32),
        pltpu.VMEM((GROUP, SUB), jnp.int32),
        pltpu.VMEM((GROUP * SUB, DIM), jnp.float32),
        pltpu.SemaphoreType.DMA,
    ],
)
def _buckle_gather(idx_hbm, off_hbm, table_hbm, out_hbm,
                   off_v, idx_v, rows_v, sem):
    wid = lax.axis_index("s") * NC + lax.axis_index("c")
    pltpu.sync_copy(off_hbm, off_v)

    def unit_body(c, carry):
        u = wid * UNITS_W + c
        f = u // BBLKS
        jb = (u % BBLKS) * GROUP
        foff = off_v[pl.ds(f, L)][0]
        pltpu.sync_copy(idx_hbm.at[f, pl.ds(jb, GROUP)], idx_v)
        for j in range(GROUP):
            for s in range(SUB // L):
                sl = pl.ds(s * L, L)
                idx_v[j, sl] = (idx_v[j, sl] + foff) * 4
        copies = [
            pltpu.async_copy(table_hbm.at[idx_v.at[j]],
                             rows_v.at[pl.ds(j * SUB, SUB)], sem)
            for j in range(GROUP)
        ]
        for cp in copies:
            cp.wait()
        pltpu.sync_copy(rows_v, out_hbm.at[f, pl.ds(jb * SUB, GROUP * SUB)])
        return carry

    lax.fori_loop(0, UNITS_W, unit_body, 0)


def kernel(categorical_inputs, embedding_weight, offsets):
    table_rm = _table_transpose(embedding_weight.T)
    idx_fm = (categorical_inputs.astype(jnp.int32).T
              .reshape(FIELDS, BATCH // SUB, SUB))
    off = jnp.pad(offsets[:FIELDS].astype(jnp.int32), (0, 3 * L - FIELDS))
    out2 = _buckle_gather(idx_fm, off, table_rm.reshape(4 * V, DIM))
    return jnp.transpose(out2, (1, 0, 2))
